# baseline (device time: 209916 ns/iter reference)
import jax
import jax.numpy as jnp
from jax import lax
from jax.experimental import pallas as pl
from jax.experimental.pallas import tpu as pltpu

N_DEV = 4
M_PER = 4096
N_PER = 1024
M_GLOBAL = N_DEV * M_PER
CHUNKS = 4
C_ROWS = M_PER // CHUNKS
DST_ORDER = (1, 3, 2)
UNITS = [(d, c) for c in range(CHUNKS) for d in DST_ORDER] + [
    (0, c) for c in range(CHUNKS)
]
N_STAGE = 4


def kernel(x):
    def body(x_ref, out_ref, stage, sendbuf, stage_sems, send_sems,
             recv_sems, copy_sems):
        my = lax.axis_index("i")

        def stage_in(u):
            d, c = UNITS[u]
            j = (my + d) % N_DEV
            cp = pltpu.make_async_copy(
                x_ref.at[pl.ds(c * C_ROWS, C_ROWS), pl.ds(j * N_PER, N_PER)],
                stage.at[u % N_STAGE],
                stage_sems.at[u % N_STAGE],
            )
            cp.start()
            return cp

        stage_q = [stage_in(u) for u in range(N_STAGE - 1)]

        barrier = pltpu.get_barrier_semaphore()
        for j in range(N_DEV):
            @pl.when(my != j)
            def _():
                pl.semaphore_signal(
                    barrier, inc=1,
                    device_id=(j,), device_id_type=pl.DeviceIdType.MESH,
                )
        pl.semaphore_wait(barrier, N_DEV - 1)

        pending = [None] * len(UNITS)
        for u, (d, c) in enumerate(UNITS):
            if u + N_STAGE - 1 < len(UNITS):
                stage_q.append(stage_in(u + N_STAGE - 1))
            stage_q.pop(0).wait()
            sendbuf[u] = stage[u % N_STAGE].astype(jnp.bfloat16)
            j = (my + d) % N_DEV
            if d == 0:
                cp = pltpu.make_async_copy(
                    sendbuf.at[u],
                    out_ref.at[pl.ds(my * M_PER + c * C_ROWS, C_ROWS), :],
                    copy_sems.at[c],
                )
                cp.start()
                pending[u] = cp
            else:
                rdma = pltpu.make_async_remote_copy(
                    src_ref=sendbuf.at[u],
                    dst_ref=out_ref.at[pl.ds(my * M_PER + c * C_ROWS, C_ROWS), :],
                    send_sem=send_sems.at[u],
                    recv_sem=recv_sems.at[my, c],
                    device_id=(j,),
                    device_id_type=pl.DeviceIdType.MESH,
                )
                rdma.start()
                pending[u] = rdma

        for u, (d, c) in enumerate(UNITS):
            if d == 0:
                pending[u].wait()
            else:
                pending[u].wait_send()
        for j in range(N_DEV):
            for c in range(CHUNKS):
                @pl.when(my != j)
                def _():
                    recv_done = pltpu.make_async_remote_copy(
                        src_ref=sendbuf.at[0],
                        dst_ref=out_ref.at[
                            pl.ds(j * M_PER + c * C_ROWS, C_ROWS), :],
                        send_sem=send_sems.at[0],
                        recv_sem=recv_sems.at[j, c],
                        device_id=(j,),
                        device_id_type=pl.DeviceIdType.MESH,
                    )
                    recv_done.wait_recv()

    n_units = len(UNITS)
    return pl.pallas_call(
        body,
        out_shape=jax.ShapeDtypeStruct((M_GLOBAL, N_PER), jnp.bfloat16),
        in_specs=[pl.BlockSpec(memory_space=pltpu.MemorySpace.HBM)],
        out_specs=pl.BlockSpec(memory_space=pltpu.MemorySpace.HBM),
        scratch_shapes=[
            pltpu.VMEM((N_STAGE, C_ROWS, N_PER), jnp.float32),
            pltpu.VMEM((n_units, C_ROWS, N_PER), jnp.bfloat16),
            pltpu.SemaphoreType.DMA((N_STAGE,)),
            pltpu.SemaphoreType.DMA((n_units,)),
            pltpu.SemaphoreType.DMA((N_DEV, CHUNKS)),
            pltpu.SemaphoreType.DMA((CHUNKS,)),
        ],
        compiler_params=pltpu.CompilerParams(
            collective_id=0, vmem_limit_bytes=56 * 1024 * 1024,
        ),
    )(x)
